# baseline (speedup 1.0000x reference)
import jax
import jax.numpy as jnp
from jax import lax
from jax.experimental import pallas as pl
from jax.experimental.pallas import tpu as pltpu
from jax.experimental.pallas import tpu_sc as plsc

NUM_USERS = 1000000
BATCH = 16384
EMBED_DIM = 64
H1 = 128
H2 = 64
BN_EPS = 1e-5

_PAIR = 2 * EMBED_DIM
_NPAIR = NUM_USERS // 2

_INFO = plsc.get_sparse_core_info()
_NC = _INFO.num_cores
_NS = _INFO.num_subcores
_NW = _NC * _NS
_ROWS_PER_W = BATCH // _NW
_CHUNK = 128
_NCHUNK = _ROWS_PER_W // _CHUNK


def _gather_body(idx_hbm, emb2_hbm, x2_hbm, idx_v, rows_v, sem):
    wid = lax.axis_index("s") * _NC + lax.axis_index("c")
    base = wid * _ROWS_PER_W
    pltpu.sync_copy(idx_hbm.at[pl.ds(base, _ROWS_PER_W)], idx_v)
    for k in range(_NCHUNK):
        pltpu.async_copy(
            emb2_hbm.at[idx_v.at[pl.ds(k * _CHUNK, _CHUNK)]],
            rows_v.at[pl.ds(k * _CHUNK, _CHUNK)],
            sem,
        )
    for k in range(_NCHUNK):
        pltpu.make_async_copy(
            emb2_hbm.at[idx_v.at[pl.ds(k * _CHUNK, _CHUNK)]],
            rows_v.at[pl.ds(k * _CHUNK, _CHUNK)],
            sem,
        ).wait()
    pltpu.sync_copy(rows_v, x2_hbm.at[pl.ds(base, _ROWS_PER_W)])


_gather = pl.kernel(
    _gather_body,
    out_type=jax.ShapeDtypeStruct((BATCH, _PAIR), jnp.float32),
    mesh=plsc.VectorSubcoreMesh(core_axis_name="c", subcore_axis_name="s"),
    scratch_types=[
        pltpu.VMEM((_ROWS_PER_W,), jnp.int32),
        pltpu.VMEM((_ROWS_PER_W, _PAIR), jnp.float32),
        pltpu.SemaphoreType.DMA,
    ],
    compiler_params=pltpu.CompilerParams(use_tc_tiling_on_sc=True),
)


_BLKB = 4096
_INV = 1.0 / (1.0 + BN_EPS) ** 0.5


def _mlp_body(x2_ref, par_ref, w1_ref, b1_ref, g1_ref, be1_ref, w2_ref,
              b2_ref, g2_ref, be2_ref, o_ref):
    x2 = x2_ref[...]
    par = par_ref[...] > 0
    x = jnp.where(par, x2[:, EMBED_DIM:], x2[:, :EMBED_DIM])
    h = jnp.dot(x, w1_ref[...], preferred_element_type=jnp.float32)
    h = h + b1_ref[...]
    h = jnp.maximum(h, 0.0)
    h = h * (_INV * g1_ref[...]) + be1_ref[...]
    o = jnp.dot(h, w2_ref[...], preferred_element_type=jnp.float32)
    o = o + b2_ref[...]
    o = jnp.maximum(o, 0.0)
    o_ref[...] = o * (_INV * g2_ref[...]) + be2_ref[...]


def _full(shape):
    return pl.BlockSpec(shape, lambda i: (0,) * len(shape))


_mlp = pl.pallas_call(
    _mlp_body,
    grid=(BATCH // _BLKB,),
    in_specs=[
        pl.BlockSpec((_BLKB, _PAIR), lambda i: (i, 0)),
        pl.BlockSpec((_BLKB, 1), lambda i: (i, 0)),
        _full((EMBED_DIM, H1)),
        _full((1, H1)),
        _full((1, H1)),
        _full((1, H1)),
        _full((H1, H2)),
        _full((1, H2)),
        _full((1, H2)),
        _full((1, H2)),
    ],
    out_specs=pl.BlockSpec((_BLKB, H2), lambda i: (i, 0)),
    out_shape=jax.ShapeDtypeStruct((BATCH, H2), jnp.float32),
)


@jax.jit
def kernel(user_ids, emb, W1, b1, g1, be1, W2, b2, g2, be2):
    idx = user_ids.astype(jnp.int32)
    emb2 = emb.reshape(_NPAIR, _PAIR)
    x2 = _gather(idx >> 1, emb2)
    return _mlp(
        x2,
        (idx & 1).reshape(BATCH, 1),
        W1,
        b1.reshape(1, H1),
        g1.reshape(1, H1),
        be1.reshape(1, H1),
        W2,
        b2.reshape(1, H2),
        g2.reshape(1, H2),
        be2.reshape(1, H2),
    )
